# Initial kernel scaffold; baseline (speedup 1.0000x reference)
import functools

import jax
import jax.numpy as jnp
from jax.experimental import pallas as pl
from jax.experimental.pallas import tpu as pltpu

E = 8
K = 2


def _router_kernel(x_ref, wg_ref, comb_ref, aux_ref):
    x = x_ref[...]
    wg = wg_ref[...]
    T = x.shape[0]
    logits = jax.lax.dot_general(
        x, wg, (((1,), (1,)), ((), ())), preferred_element_type=jnp.float32
    )
    m = jnp.max(logits, axis=-1, keepdims=True)
    ex = jnp.exp(logits - m)
    denom = jnp.sum(ex, axis=-1, keepdims=True)
    probs = ex / denom

    m1 = jnp.max(probs, axis=-1, keepdims=True)
    is1 = probs >= m1
    first = is1 & (jnp.cumsum(is1.astype(jnp.int32), axis=-1) == 1)
    probs2 = jnp.where(first, -jnp.inf, probs)
    m2 = jnp.max(probs2, axis=-1, keepdims=True)
    is2 = probs2 >= m2
    second = is2 & (jnp.cumsum(is2.astype(jnp.int32), axis=-1) == 1)
    rden = 1.0 / (m1[:, 0] + m2[:, 0] + 1e-9)
    w1 = m1[:, 0] * rden
    w2 = m2[:, 0] * rden
    comb = first.astype(jnp.float32) * w1[:, None] + \
        second.astype(jnp.float32) * w2[:, None]
    comb_ref[...] = comb

    sel = first.astype(jnp.float32) + second.astype(jnp.float32)
    tokens_per_expert = jnp.sum(sel, axis=0) / (T * K + 1e-9)
    avg_probs = jnp.mean(probs, axis=0)
    load_balance = E * jnp.sum(tokens_per_expert * avg_probs)
    lse = jnp.log(denom[:, 0]) + m[:, 0]
    z_loss = jnp.mean(jnp.square(lse)) * 0.001
    entropy = jnp.mean(-jnp.sum(probs * jnp.log(probs + 1e-9), axis=-1))
    entropy_loss = (jnp.log(jnp.float32(E)) - entropy) * 0.01
    usage = jnp.mean((tokens_per_expert > 0.01).astype(jnp.float32))
    utilization_loss = (1.0 - usage) * 0.1
    aux_ref[0, 0] = load_balance + z_loss + entropy_loss + utilization_loss


def _expert_kernel(x_ref, wgt_ref, wut_ref, wdt_ref, comb_ref, out_ref):
    e = pl.program_id(1)

    @pl.when(e == 0)
    def _():
        out_ref[...] = jnp.zeros_like(out_ref)

    x = x_ref[...]
    gate = jnp.dot(x, wgt_ref[0], preferred_element_type=jnp.float32)
    up = jnp.dot(x, wut_ref[0], preferred_element_type=jnp.float32)
    h = (gate * jax.lax.logistic(gate) * up).astype(jnp.bfloat16)
    out_e = jnp.dot(h, wdt_ref[0], preferred_element_type=jnp.float32)
    out_ref[...] += comb_ref[0][:, None] * out_e


def kernel(hidden_states, W_gate, Wg, Wu, Wd, Wsg, Wsu, Wsd, shared_gate):
    b, s, h = hidden_states.shape
    x = hidden_states.reshape(-1, h)
    T = x.shape[0]
    I = Wg.shape[1]
    NE = E + 1

    comb, aux = pl.pallas_call(
        _router_kernel,
        out_shape=(
            jax.ShapeDtypeStruct((T, E), jnp.float32),
            jax.ShapeDtypeStruct((1, 1), jnp.float32),
        ),
    )(x, W_gate)

    wgt = jnp.concatenate([Wg, Wsg[None]], 0).transpose(0, 2, 1)
    wut = jnp.concatenate([Wu, Wsu[None]], 0).transpose(0, 2, 1)
    wdt = jnp.concatenate([Wd, Wsd[None]], 0).transpose(0, 2, 1)
    shared_w = jnp.broadcast_to(
        jax.nn.sigmoid(shared_gate)[None, :], (T, 1)).astype(jnp.float32)
    comb9_t = jnp.concatenate([comb, shared_w], 1).T

    BT = 1024
    NT = T // BT
    out = pl.pallas_call(
        _expert_kernel,
        grid=(NT, NE),
        in_specs=[
            pl.BlockSpec((BT, h), lambda t, e: (t, 0)),
            pl.BlockSpec((1, h, I), lambda t, e: (e, 0, 0)),
            pl.BlockSpec((1, h, I), lambda t, e: (e, 0, 0)),
            pl.BlockSpec((1, I, h), lambda t, e: (e, 0, 0)),
            pl.BlockSpec((1, BT), lambda t, e: (e, t)),
        ],
        out_specs=pl.BlockSpec((BT, h), lambda t, e: (t, 0)),
        out_shape=jax.ShapeDtypeStruct((T, h), jnp.float32),
    )(
        x.astype(jnp.bfloat16),
        wgt.astype(jnp.bfloat16),
        wut.astype(jnp.bfloat16),
        wdt.astype(jnp.bfloat16),
        comb9_t,
    )

    return out.reshape(b, s, h), aux[0, 0]


# speedup vs baseline: 1.1869x; 1.1869x over previous
import functools

import jax
import jax.numpy as jnp
from jax.experimental import pallas as pl
from jax.experimental.pallas import tpu as pltpu

E = 8
K = 2


def _router_kernel(x_ref, wg_ref, comb_ref, aux_ref):
    x = x_ref[...]
    wg = wg_ref[...]
    T = x.shape[0]
    logits = jax.lax.dot_general(
        x, wg, (((1,), (1,)), ((), ())), preferred_element_type=jnp.float32
    )
    m = jnp.max(logits, axis=-1, keepdims=True)
    ex = jnp.exp(logits - m)
    denom = jnp.sum(ex, axis=-1, keepdims=True)
    probs = ex / denom

    ii = jax.lax.broadcasted_iota(jnp.int32, (E, E), 0)
    jj = jax.lax.broadcasted_iota(jnp.int32, (E, E), 1)
    strict = (ii < jj).astype(jnp.float32)
    m1 = jnp.max(probs, axis=-1, keepdims=True)
    is1 = probs >= m1
    prior1 = jnp.dot(is1.astype(jnp.float32), strict,
                     preferred_element_type=jnp.float32)
    first = is1 & (prior1 == 0.0)
    probs2 = jnp.where(first, -jnp.inf, probs)
    m2 = jnp.max(probs2, axis=-1, keepdims=True)
    is2 = probs2 >= m2
    prior2 = jnp.dot(is2.astype(jnp.float32), strict,
                     preferred_element_type=jnp.float32)
    second = is2 & (prior2 == 0.0)
    rden = 1.0 / (m1[:, 0] + m2[:, 0] + 1e-9)
    w1 = m1[:, 0] * rden
    w2 = m2[:, 0] * rden
    comb = first.astype(jnp.float32) * w1[:, None] + \
        second.astype(jnp.float32) * w2[:, None]
    comb_ref[...] = comb

    sel = first.astype(jnp.float32) + second.astype(jnp.float32)
    tokens_per_expert = jnp.sum(sel, axis=0) / (T * K + 1e-9)
    avg_probs = jnp.mean(probs, axis=0)
    load_balance = E * jnp.sum(tokens_per_expert * avg_probs)
    lse = jnp.log(denom[:, 0]) + m[:, 0]
    z_loss = jnp.mean(jnp.square(lse)) * 0.001
    entropy = jnp.mean(-jnp.sum(probs * jnp.log(probs + 1e-9), axis=-1))
    entropy_loss = (jnp.log(jnp.float32(E)) - entropy) * 0.01
    usage = jnp.mean((tokens_per_expert > 0.01).astype(jnp.float32))
    utilization_loss = (1.0 - usage) * 0.1
    total = load_balance + z_loss + entropy_loss + utilization_loss
    aux_ref[...] = total[None, None]


def _expert_kernel(x_ref, wgt_ref, wut_ref, wdt_ref, comb_ref, out_ref):
    e = pl.program_id(1)

    @pl.when(e == 0)
    def _():
        out_ref[...] = jnp.zeros_like(out_ref)

    x = x_ref[...]
    gate = jnp.dot(x, wgt_ref[0], preferred_element_type=jnp.float32)
    up = jnp.dot(x, wut_ref[0], preferred_element_type=jnp.float32)
    h = (gate * jax.lax.logistic(gate) * up).astype(jnp.bfloat16)
    out_e = jnp.dot(h, wdt_ref[0], preferred_element_type=jnp.float32)
    out_ref[...] += comb_ref[0, 0][:, None] * out_e


def kernel(hidden_states, W_gate, Wg, Wu, Wd, Wsg, Wsu, Wsd, shared_gate):
    b, s, h = hidden_states.shape
    x = hidden_states.reshape(-1, h)
    T = x.shape[0]
    I = Wg.shape[1]
    NE = E + 1

    comb, aux = pl.pallas_call(
        _router_kernel,
        out_shape=(
            jax.ShapeDtypeStruct((T, E), jnp.float32),
            jax.ShapeDtypeStruct((1, 1), jnp.float32),
        ),
    )(x, W_gate)

    wgt = jnp.concatenate([Wg, Wsg[None]], 0).transpose(0, 2, 1)
    wut = jnp.concatenate([Wu, Wsu[None]], 0).transpose(0, 2, 1)
    wdt = jnp.concatenate([Wd, Wsd[None]], 0).transpose(0, 2, 1)
    shared_w = jnp.broadcast_to(
        jax.nn.sigmoid(shared_gate)[None, :], (T, 1)).astype(jnp.float32)
    comb9_t = jnp.concatenate([comb, shared_w], 1).T.reshape(NE, 1, T)

    BT = 1024
    NT = T // BT
    out = pl.pallas_call(
        _expert_kernel,
        grid=(NT, NE),
        in_specs=[
            pl.BlockSpec((BT, h), lambda t, e: (t, 0)),
            pl.BlockSpec((1, h, I), lambda t, e: (e, 0, 0)),
            pl.BlockSpec((1, h, I), lambda t, e: (e, 0, 0)),
            pl.BlockSpec((1, I, h), lambda t, e: (e, 0, 0)),
            pl.BlockSpec((1, 1, BT), lambda t, e: (e, 0, t)),
        ],
        out_specs=pl.BlockSpec((BT, h), lambda t, e: (t, 0)),
        out_shape=jax.ShapeDtypeStruct((T, h), jnp.float32),
    )(
        x.astype(jnp.bfloat16),
        wgt.astype(jnp.bfloat16),
        wut.astype(jnp.bfloat16),
        wdt.astype(jnp.bfloat16),
        comb9_t,
    )

    return out.reshape(b, s, h), aux[0, 0]


# speedup vs baseline: 1.8602x; 1.5673x over previous
import functools

import jax
import jax.numpy as jnp
from jax.experimental import pallas as pl
from jax.experimental.pallas import tpu as pltpu

E = 8
K = 2


def _router_kernel(x_ref, wg_ref, comb_ref, aux_ref):
    x = x_ref[...]
    wg = wg_ref[...]
    T = x.shape[0]
    logits = jax.lax.dot_general(
        x, wg, (((1,), (1,)), ((), ())), preferred_element_type=jnp.float32
    )
    m = jnp.max(logits, axis=-1, keepdims=True)
    ex = jnp.exp(logits - m)
    denom = jnp.sum(ex, axis=-1, keepdims=True)
    probs = ex / denom

    ii = jax.lax.broadcasted_iota(jnp.int32, (E, E), 0)
    jj = jax.lax.broadcasted_iota(jnp.int32, (E, E), 1)
    strict = (ii < jj).astype(jnp.float32)
    m1 = jnp.max(probs, axis=-1, keepdims=True)
    is1 = probs >= m1
    prior1 = jnp.dot(is1.astype(jnp.float32), strict,
                     preferred_element_type=jnp.float32)
    first = is1 & (prior1 == 0.0)
    probs2 = jnp.where(first, -jnp.inf, probs)
    m2 = jnp.max(probs2, axis=-1, keepdims=True)
    is2 = probs2 >= m2
    prior2 = jnp.dot(is2.astype(jnp.float32), strict,
                     preferred_element_type=jnp.float32)
    second = is2 & (prior2 == 0.0)
    rden = 1.0 / (m1[:, 0] + m2[:, 0] + 1e-9)
    w1 = m1[:, 0] * rden
    w2 = m2[:, 0] * rden
    comb = first.astype(jnp.float32) * w1[:, None] + \
        second.astype(jnp.float32) * w2[:, None]
    comb_ref[...] = comb

    sel = first.astype(jnp.float32) + second.astype(jnp.float32)
    tokens_per_expert = jnp.sum(sel, axis=0) / (T * K + 1e-9)
    avg_probs = jnp.mean(probs, axis=0)
    load_balance = E * jnp.sum(tokens_per_expert * avg_probs)
    lse = jnp.log(denom[:, 0]) + m[:, 0]
    z_loss = jnp.mean(jnp.square(lse)) * 0.001
    entropy = jnp.mean(-jnp.sum(probs * jnp.log(probs + 1e-9), axis=-1))
    entropy_loss = (jnp.log(jnp.float32(E)) - entropy) * 0.01
    usage = jnp.mean((tokens_per_expert > 0.01).astype(jnp.float32))
    utilization_loss = (1.0 - usage) * 0.1
    total = load_balance + z_loss + entropy_loss + utilization_loss
    aux_ref[...] = total[None, None]


def _ffn_block(x, wg, wu, wd):
    dn = (((1,), (1,)), ((), ()))
    gate = jax.lax.dot_general(x, wg, dn, preferred_element_type=jnp.float32)
    up = jax.lax.dot_general(x, wu, dn, preferred_element_type=jnp.float32)
    h = (gate * jax.lax.logistic(gate) * up).astype(jnp.bfloat16)
    return jax.lax.dot_general(h, wd, dn, preferred_element_type=jnp.float32)


def _expert_kernel(x_ref, wg_ref, wu_ref, wd_ref, wsg_ref, wsu_ref,
                   wsd_ref, comb_ref, out_ref):
    i = pl.program_id(0)
    e = pl.program_id(1)

    @pl.when((i == 0) & (e == 0))
    def _():
        out_ref[...] = jnp.zeros_like(out_ref)

    x = x_ref[...]
    w_row = comb_ref[0, 0][:, None]

    @pl.when(e < E)
    def _():
        o = _ffn_block(x,
                       wg_ref[0].astype(jnp.bfloat16),
                       wu_ref[0].astype(jnp.bfloat16),
                       wd_ref[0].astype(jnp.bfloat16))
        out_ref[...] += w_row * o

    @pl.when(e == E)
    def _():
        o = _ffn_block(x,
                       wsg_ref[...].astype(jnp.bfloat16),
                       wsu_ref[...].astype(jnp.bfloat16),
                       wsd_ref[...].astype(jnp.bfloat16))
        out_ref[...] += w_row * o


def kernel(hidden_states, W_gate, Wg, Wu, Wd, Wsg, Wsu, Wsd, shared_gate):
    b, s, h = hidden_states.shape
    x = hidden_states.reshape(-1, h)
    T = x.shape[0]
    I = Wg.shape[1]
    NE = E + 1

    comb, aux = pl.pallas_call(
        _router_kernel,
        out_shape=(
            jax.ShapeDtypeStruct((T, E), jnp.float32),
            jax.ShapeDtypeStruct((1, 1), jnp.float32),
        ),
    )(x, W_gate)

    shared_w = jnp.broadcast_to(
        jax.nn.sigmoid(shared_gate)[None, :], (T, 1)).astype(jnp.float32)
    comb9_t = jnp.concatenate([comb, shared_w], 1).T.reshape(NE, 1, T)

    BI = 768
    NI = I // BI
    rt_ih = lambda i, e: (jnp.minimum(e, E - 1), i, 0)
    out = pl.pallas_call(
        _expert_kernel,
        grid=(NI, NE),
        in_specs=[
            pl.BlockSpec((T, h), lambda i, e: (0, 0)),
            pl.BlockSpec((1, BI, h), rt_ih),
            pl.BlockSpec((1, BI, h), rt_ih),
            pl.BlockSpec((1, h, BI), lambda i, e: (jnp.minimum(e, E - 1), 0, i)),
            pl.BlockSpec((BI, h), lambda i, e: (i, 0)),
            pl.BlockSpec((BI, h), lambda i, e: (i, 0)),
            pl.BlockSpec((h, BI), lambda i, e: (0, i)),
            pl.BlockSpec((1, 1, T), lambda i, e: (e, 0, 0)),
        ],
        out_specs=pl.BlockSpec((T, h), lambda i, e: (0, 0)),
        out_shape=jax.ShapeDtypeStruct((T, h), jnp.float32),
    )(
        x.astype(jnp.bfloat16),
        Wg, Wu, Wd, Wsg, Wsu, Wsd,
        comb9_t,
    )

    return out.reshape(b, s, h), aux[0, 0]
